# baseline (speedup 1.0000x reference)
import functools
import jax
import jax.numpy as jnp
from jax import lax
from jax.experimental import pallas as pl
from jax.experimental.pallas import tpu as pltpu
from jax.experimental.pallas import tpu_sc as plsc

XY_GRID = (432, 496)
XY_R = ((0.0, 69.12), (-39.68, 39.68), (-3.0, 1.0))
CYL_GRID = (2560, 100)
CYL_R = ((-3.141592653589793, 3.141592653589793), (-3.0, 1.0), (0.0, 69.12))
NPTS = 30000
B = 2

NP_PAD = 30720
BN_ROWS = 3072
N_BLOCKS = NP_PAD // BN_ROWS
M_PTS = B * NP_PAD

try:
    _SC = plsc.get_sparse_core_info()
    NC, NS, L = _SC.num_cores, _SC.num_subcores, _SC.num_lanes
except Exception:
    NC, NS, L = 2, 16, 16
NW = NC * NS
CH = 128
_SC_PARAMS = pltpu.CompilerParams(use_tc_tiling_on_sc=False)


def _pad_pts(x, value=0.0):
    pad = [(0, 0)] * x.ndim
    pad[1] = (0, NP_PAD - x.shape[1])
    return jnp.pad(x, pad, constant_values=value)


def _pad_to(x, n, axis, value=0.0):
    pad = [(0, 0)] * x.ndim
    pad[axis] = (0, n - x.shape[axis])
    return jnp.pad(x, pad, constant_values=value)


KQ = 5


@functools.lru_cache(maxsize=None)
def _sc_gather_rows_fn(t_rows, d, m):
    b_per_w = m // NW
    n_inner = b_per_w // (KQ * CH)
    mesh = plsc.VectorSubcoreMesh(core_axis_name="c", subcore_axis_name="s")

    @functools.partial(
        pl.kernel, mesh=mesh,
        out_type=jax.ShapeDtypeStruct((m, d), jnp.float32),
        compiler_params=_SC_PARAMS,
        scratch_types=[
            pltpu.VMEM((KQ, CH), jnp.int32),
            pltpu.VMEM((KQ * CH, d), jnp.float32),
            pltpu.SemaphoreType.DMA,
        ],
    )
    def k(table_hbm, idx_hbm, out_hbm, idx_v, rows_v, sem):
        wid = lax.axis_index("s") * NC + lax.axis_index("c")
        base = wid * b_per_w

        def body(g, _):
            off = base + g * (KQ * CH)
            pltpu.sync_copy(idx_hbm.at[pl.ds(off // CH, KQ)], idx_v)
            cps = [pltpu.async_copy(table_hbm.at[idx_v.at[q]],
                                    rows_v.at[pl.ds(q * CH, CH)], sem)
                   for q in range(KQ)]
            for cp in cps:
                cp.wait()
            pltpu.sync_copy(rows_v, out_hbm.at[pl.ds(off, KQ * CH)])
            return 0

        lax.fori_loop(0, n_inner, body, 0)

    return k


def _sc_gather_rows(table, idx):
    m = idx.shape[0]
    return _sc_gather_rows_fn(table.shape[0], table.shape[1], m)(
        table, idx.reshape(m // CH, CH))


@functools.lru_cache(maxsize=None)
def _sc_gather2_fn(s_len, t_rows, d, m):
    b_per_w = m // NW
    n_inner = b_per_w // (KQ * CH)
    mesh = plsc.VectorSubcoreMesh(core_axis_name="c", subcore_axis_name="s")

    @functools.partial(
        pl.kernel, mesh=mesh,
        out_type=jax.ShapeDtypeStruct((m, d), jnp.float32),
        compiler_params=_SC_PARAMS,
        scratch_types=[
            pltpu.VMEM((KQ, CH), jnp.int32),
            pltpu.VMEM((KQ, CH), jnp.int32),
            pltpu.VMEM((KQ * CH, d), jnp.float32),
            pltpu.SemaphoreType.DMA,
        ],
    )
    def k(s_hbm, table_hbm, idx_hbm, out_hbm, idx_v, slot_v, rows_v, sem):
        wid = lax.axis_index("s") * NC + lax.axis_index("c")
        base = wid * b_per_w

        def body(g, _):
            off = base + g * (KQ * CH)
            pltpu.sync_copy(idx_hbm.at[pl.ds(off // CH, KQ)], idx_v)
            cps = [pltpu.async_copy(s_hbm.at[idx_v.at[q]], slot_v.at[q], sem)
                   for q in range(KQ)]
            for cp in cps:
                cp.wait()
            cps = [pltpu.async_copy(table_hbm.at[slot_v.at[q]],
                                    rows_v.at[pl.ds(q * CH, CH)], sem)
                   for q in range(KQ)]
            for cp in cps:
                cp.wait()
            pltpu.sync_copy(rows_v, out_hbm.at[pl.ds(off, KQ * CH)])
            return 0

        lax.fori_loop(0, n_inner, body, 0)

    return k


def _sc_gather2(s_map, table, idx):
    m = idx.shape[0]
    return _sc_gather2_fn(s_map.shape[0], table.shape[0], table.shape[1],
                          m)(s_map, table, idx.reshape(m // CH, CH))


@functools.lru_cache(maxsize=None)
def _sc_scatter_add_fn(s_len, m):
    half = m // NC
    b_per_w = half // NS
    n_inner = b_per_w // CH
    tile_len = s_len // NS
    n_ms = tile_len // (16 * CH)
    mesh = plsc.VectorSubcoreMesh(core_axis_name="c", subcore_axis_name="s")

    @functools.partial(
        pl.kernel, mesh=mesh,
        out_type=jax.ShapeDtypeStruct((NC, s_len), jnp.int32),
        compiler_params=_SC_PARAMS,
        scratch_types=[
            pltpu.VMEM_SHARED((s_len,), jnp.int32),
            pltpu.VMEM((16 * CH,), jnp.int32),
            pltpu.VMEM((CH,), jnp.int32),
            pltpu.VMEM((CH,), jnp.int32),
        ],
    )
    def k(idx_hbm, val_hbm, out_hbm, smem, zbuf, idx_v, val_v):
        c = lax.axis_index("c")
        s = lax.axis_index("s")

        def zb(i, _):
            zbuf[pl.ds(i * L, L)] = jnp.zeros((L,), jnp.int32)
            return 0

        lax.fori_loop(0, (16 * CH) // L, zb, 0)
        tbase = s * tile_len

        def ms(i, _):
            pltpu.sync_copy(zbuf, smem.at[pl.ds(tbase + i * 16 * CH, 16 * CH)])
            return 0

        lax.fori_loop(0, n_ms, ms, 0)
        plsc.subcore_barrier()

        base = c * half + s * b_per_w

        def body(g, _):
            off = base + g * CH
            pltpu.sync_copy(idx_hbm.at[pl.ds(off, CH)], idx_v)
            pltpu.sync_copy(val_hbm.at[pl.ds(off, CH)], val_v)
            pltpu.sync_copy(val_v, smem.at[idx_v], add=True)
            return 0

        lax.fori_loop(0, n_inner, body, 0)
        plsc.subcore_barrier()

        def wr(i, _):
            pltpu.sync_copy(smem.at[pl.ds(tbase + i * 16 * CH, 16 * CH)],
                            zbuf)
            pltpu.sync_copy(zbuf, out_hbm.at[c, pl.ds(tbase + i * 16 * CH,
                                                      16 * CH)])
            return 0

        lax.fori_loop(0, n_ms, wr, 0)

    return k


N_EMPTY = 64


def _sc_scatter_add(idx, vals, s_len):
    parts = _sc_scatter_add_fn(s_len, idx.shape[0])(idx, vals)
    s_raw = parts[0] + parts[1]
    spread = jnp.arange(s_len, dtype=jnp.int32) & (N_EMPTY - 1)
    return jnp.where(s_raw == 0, spread, s_raw - 1 + N_EMPTY)


def _shift_down(x, s, fill):
    top = jnp.full((s,) + x.shape[1:], fill, x.dtype)
    return jnp.concatenate([top, x[: x.shape[0] - s]], axis=0)


def _seg_scan_body(is_max, c, ks_ref, kp_ref, kn_ref, v_ref, tot_ref, e_ref,
                   sval_ref, scell_ref, carry_ref, *, s_stride, nv):
    b = pl.program_id(0)
    j = pl.program_id(1)
    ks = ks_ref[0, 0, 0][:, None]
    kp = kp_ref[0, 0, 0][:, None]
    ident = jnp.float32(-1e30) if is_max else jnp.float32(0.0)
    vals = v_ref[0]
    n = vals.shape[0]
    s = 1
    while s < n:
        ks_s = _shift_down(ks, s, -2)
        v_s = _shift_down(vals, s, ident)
        gate = ks == ks_s
        if is_max:
            vals = jnp.maximum(vals, jnp.where(gate, v_s, ident))
        else:
            vals = vals + jnp.where(gate, v_s, 0.0)
        s *= 2

    @pl.when(j == 0)
    def _():
        carry_ref[...] = jnp.full_like(carry_ref, ident)

    kp_first = kp_ref[0, 0, 0, 0]
    span = ks == kp_first
    carry = carry_ref[0, :][None, :]
    if is_max:
        vals = jnp.where(span, jnp.maximum(vals, carry), vals)
    else:
        vals = jnp.where(span, vals + carry, vals)
    carry_ref[0, :] = vals[n - 1, :]
    tot_ref[0] = vals
    if e_ref is not None:
        kn = kn_ref[0, 0, 0]
        e = ks_ref[0, 0, 0] != kn
        rowg = b * NP_PAD + j * n + lax.broadcasted_iota(jnp.int32, (n,), 0)
        e_ref[0, 0, 0] = e.astype(jnp.float32)
        sval_ref[0, 0, 0] = jnp.where(e, rowg + 1, 0)
        scell_ref[0, 0, 0] = b * s_stride + ks_ref[0, 0, 0]


def _seg_scan(ks4, kp4, kn4, vals, is_max, with_slots, s_stride=0, nv=0):
    c = vals.shape[-1]
    outs = [jax.ShapeDtypeStruct((B, NP_PAD, c), jnp.float32)]
    out_specs = [pl.BlockSpec((1, BN_ROWS, c), lambda b, j: (b, j, 0))]
    if with_slots:
        for dt in (jnp.float32, jnp.int32, jnp.int32):
            outs.append(jax.ShapeDtypeStruct((B, N_BLOCKS, 1, BN_ROWS), dt))
            out_specs.append(
                pl.BlockSpec((1, 1, 1, BN_ROWS), lambda b, j: (b, j, 0, 0)))
        body = functools.partial(_seg_scan_body, is_max, c,
                                 s_stride=s_stride, nv=nv)
    else:
        def body(ks_ref, kp_ref, kn_ref, v_ref, tot_ref, carry_ref):
            _seg_scan_body(is_max, c, ks_ref, kp_ref, kn_ref, v_ref, tot_ref,
                           None, None, None, carry_ref, s_stride=0, nv=0)
    res = pl.pallas_call(
        body,
        grid=(B, N_BLOCKS),
        in_specs=[
            pl.BlockSpec((1, 1, 1, BN_ROWS), lambda b, j: (b, j, 0, 0)),
            pl.BlockSpec((1, 1, 1, BN_ROWS), lambda b, j: (b, j, 0, 0)),
            pl.BlockSpec((1, 1, 1, BN_ROWS), lambda b, j: (b, j, 0, 0)),
            pl.BlockSpec((1, BN_ROWS, c), lambda b, j: (b, j, 0)),
        ],
        out_specs=out_specs if with_slots else out_specs[0],
        out_shape=outs if with_slots else outs[0],
        scratch_shapes=[pltpu.VMEM((1, c), jnp.float32)],
    )(ks4, kp4, kn4, vals)
    return res


R24 = BN_ROWS // 128


def _prep_body(p_ref, mask_ref, key_ref, valid_ref, cen_ref, pay_ref,
               idxg_ref, ids4_ref, w4_ref, *, grid, ranges, s_stride):
    gx, gy = grid
    (x0, x1), (y0, y1), (z0, z1) = ranges
    b = pl.program_id(0)
    vsx = (x1 - x0) / gx
    vsy = (y1 - y0) / gy
    vsz = (z1 - z0) / 1.0
    px = p_ref[0, 0, 0]
    py = p_ref[0, 0, 1]
    pz = p_ref[0, 0, 2]
    mask = mask_ref[0, 0]
    fx = (px - x0) / vsx
    fy = (py - y0) / vsy
    fz = (pz - z0) / vsz
    cx = jnp.floor(fx)
    cy = jnp.floor(fy)
    cz = jnp.floor(fz)
    inr = ((cx >= 0) & (cx < gx) & (cy >= 0) & (cy < gy)
           & (cz >= 0) & (cz < 1))
    valid = inr & (mask > 0.5)
    ccx = jnp.clip(cx, 0, gx - 1)
    ccy = jnp.clip(cy, 0, gy - 1)
    idx = (ccx * gy + ccy).astype(jnp.int32)
    nv = gx * gy
    key = jnp.where(valid, idx, nv)
    key = jnp.where(mask > -0.5, key, nv + 1)
    key_ref[0, 0] = key
    valid_ref[0, 0] = valid.astype(jnp.float32)
    cen_ref[0, 0, 0] = (ccx + 0.5) * vsx + x0
    cen_ref[0, 0, 1] = (ccy + 0.5) * vsy + y0
    cen_ref[0, 0, 2] = jnp.full_like(px, 0.5 * vsz + z0)
    one = jnp.ones_like(px)
    zero = jnp.zeros_like(px)
    for c, v in enumerate([one, px, py, pz, px * px, py * py, pz * pz,
                           px * py, px * pz, py * pz,
                           fx, fy, zero, zero, zero, zero]):
        pay_ref[0, 0, c] = v
    idxg_ref[0, 0] = b * s_stride + idx
    xq0 = jnp.floor(fx)
    yq0 = jnp.floor(fy)
    x0c = jnp.clip(xq0, 0, gx - 1).astype(jnp.int32)
    x1c = jnp.clip(xq0 + 1, 0, gx - 1).astype(jnp.int32)
    y0c = jnp.clip(yq0, 0, gy - 1).astype(jnp.int32)
    y1c = jnp.clip(yq0 + 1, 0, gy - 1).astype(jnp.int32)
    base = b * s_stride
    ids4_ref[0, 0, 0] = base + x0c * gy + y0c
    ids4_ref[0, 0, 1] = base + x0c * gy + y1c
    ids4_ref[0, 0, 2] = base + x1c * gy + y0c
    ids4_ref[0, 0, 3] = base + x1c * gy + y1c
    wxa = (xq0 + 1.0) - fx
    wxc = fx - xq0
    wya = (yq0 + 1.0) - fy
    wyb = fy - yq0
    realrow = (mask > -0.5).astype(jnp.float32)
    w4_ref[0, 0, 0] = wxa * wya * realrow
    w4_ref[0, 0, 1] = wxa * wyb * realrow
    w4_ref[0, 0, 2] = wxc * wya * realrow
    w4_ref[0, 0, 3] = wxc * wyb * realrow


def _prep(p_planes, mask_planes, grid, ranges):
    gx, gy = grid
    s_stride = gx * gy + 2
    body = functools.partial(_prep_body, grid=grid, ranges=ranges,
                             s_stride=s_stride)

    def pm(b, j):
        return (b, j, 0, 0)

    def pm5(b, j):
        return (b, j, 0, 0, 0)

    return pl.pallas_call(
        body,
        grid=(B, N_BLOCKS),
        in_specs=[
            pl.BlockSpec((1, 1, 3, R24, 128), pm5),
            pl.BlockSpec((1, 1, R24, 128), pm),
        ],
        out_specs=[
            pl.BlockSpec((1, 1, R24, 128), pm),
            pl.BlockSpec((1, 1, R24, 128), pm),
            pl.BlockSpec((1, 1, 3, R24, 128), pm5),
            pl.BlockSpec((1, 1, 16, R24, 128), pm5),
            pl.BlockSpec((1, 1, R24, 128), pm),
            pl.BlockSpec((1, 1, 4, R24, 128), pm5),
            pl.BlockSpec((1, 1, 4, R24, 128), pm5),
        ],
        out_shape=[
            jax.ShapeDtypeStruct((B, N_BLOCKS, R24, 128), jnp.int32),
            jax.ShapeDtypeStruct((B, N_BLOCKS, R24, 128), jnp.float32),
            jax.ShapeDtypeStruct((B, N_BLOCKS, 3, R24, 128), jnp.float32),
            jax.ShapeDtypeStruct((B, N_BLOCKS, 16, R24, 128), jnp.float32),
            jax.ShapeDtypeStruct((B, N_BLOCKS, R24, 128), jnp.int32),
            jax.ShapeDtypeStruct((B, N_BLOCKS, 4, R24, 128), jnp.int32),
            jax.ShapeDtypeStruct((B, N_BLOCKS, 4, R24, 128), jnp.float32),
        ],
    )(p_planes, mask_planes)


def _stats_finalize_body(g_ref, p_ref, valid_ref, rm_ref, cen_ref, out_ref):
    m = valid_ref[0, 0]
    realrow = (rm_ref[0, 0] > -0.5).astype(jnp.float32)
    mr = m * realrow
    cnt = g_ref[0, 0, 0]
    cntc = jnp.maximum(cnt, 1.0)
    rc = 1.0 / cntc
    p3 = [p_ref[0, 0, c] for c in range(3)]
    mean = [g_ref[0, 0, 1 + c] * rc for c in range(3)]
    for c in range(3):
        out_ref[0, 0, c] = (p3[c] - cen_ref[0, 0, c]) * realrow
    out_ref[0, 0, 3] = cnt * mr
    for c in range(3):
        out_ref[0, 0, 4 + c] = (p3[c] - mean[c]) * mr
    e2 = [g_ref[0, 0, 4 + c] * rc for c in range(6)]
    cov = [e2[0] - mean[0] * mean[0], e2[3] - mean[0] * mean[1],
           e2[4] - mean[0] * mean[2],
           e2[3] - mean[0] * mean[1], e2[1] - mean[1] * mean[1],
           e2[5] - mean[1] * mean[2],
           e2[4] - mean[0] * mean[2], e2[5] - mean[1] * mean[2],
           e2[2] - mean[2] * mean[2]]
    for c in range(9):
        out_ref[0, 0, 7 + c] = cov[c] * mr
    for c in range(3):
        out_ref[0, 0, 16 + c] = mean[c] * mr


def _stats_finalize(g_planes, p_planes, valid_pl, rm_pl, cen_planes):
    def pm(b, j):
        return (b, j, 0, 0)

    def pm5(b, j):
        return (b, j, 0, 0, 0)

    return pl.pallas_call(
        _stats_finalize_body,
        grid=(B, N_BLOCKS),
        in_specs=[
            pl.BlockSpec((1, 1, 16, R24, 128), pm5),
            pl.BlockSpec((1, 1, 3, R24, 128), pm5),
            pl.BlockSpec((1, 1, R24, 128), pm),
            pl.BlockSpec((1, 1, R24, 128), pm),
            pl.BlockSpec((1, 1, 3, R24, 128), pm5),
        ],
        out_specs=pl.BlockSpec((1, 1, 19, R24, 128), pm5),
        out_shape=jax.ShapeDtypeStruct((B, N_BLOCKS, 19, R24, 128),
                                       jnp.float32),
    )(g_planes, p_planes, valid_pl, rm_pl, cen_planes)


def _mm_stats_body(x_ref, wt_ref, h_ref, stats_ref, acc_ref):
    b = pl.program_id(0)
    j = pl.program_id(1)

    @pl.when(jnp.logical_and(b == 0, j == 0))
    def _():
        acc_ref[...] = jnp.zeros_like(acc_ref)

    h = jnp.dot(x_ref[0], wt_ref[...], preferred_element_type=jnp.float32)
    h_ref[0] = h
    acc_ref[0, :] += jnp.sum(h, axis=0)
    acc_ref[1, :] += jnp.sum(h * h, axis=0)

    @pl.when(jnp.logical_and(b == B - 1, j == N_BLOCKS - 1))
    def _():
        stats_ref[...] = acc_ref[...]


def _pn_matmul_stats(x, wt):
    cp = x.shape[-1]
    return pl.pallas_call(
        _mm_stats_body,
        grid=(B, N_BLOCKS),
        in_specs=[
            pl.BlockSpec((1, BN_ROWS, cp), lambda b, j: (b, j, 0)),
            pl.BlockSpec((cp, 64), lambda b, j: (0, 0)),
        ],
        out_specs=[
            pl.BlockSpec((1, BN_ROWS, 64), lambda b, j: (b, j, 0)),
            pl.BlockSpec((2, 64), lambda b, j: (0, 0)),
        ],
        out_shape=[
            jax.ShapeDtypeStruct((B, NP_PAD, 64), jnp.float32),
            jax.ShapeDtypeStruct((2, 64), jnp.float32),
        ],
        scratch_shapes=[pltpu.VMEM((2, 64), jnp.float32)],
    )(x, wt)


def _norm_body(h_ref, stats_ref, gb_ref, mask_ref, h2_ref, papre_ref, ca_ref,
               ca_acc):
    b = pl.program_id(0)
    j = pl.program_id(1)
    cnt = float(B * NPTS)
    mu = stats_ref[0, :] / cnt
    var = stats_ref[1, :] / cnt - mu * mu
    gamma = gb_ref[0, :]
    beta = gb_ref[1, :]
    h = h_ref[0]
    h2 = (h - mu[None, :]) * jax.lax.rsqrt(var + 1e-3)[None, :]
    h2 = h2 * gamma[None, :] + beta[None, :]
    h2 = jnp.maximum(h2, 0.0) * mask_ref[0, 0, 0][:, None]
    h2_ref[0] = h2
    papre_ref[0, 0, 0, :] = jnp.max(h2, axis=1)
    blk_ca = jnp.max(h2, axis=0, keepdims=True)

    @pl.when(j == 0)
    def _():
        ca_acc[pl.ds(b, 1), :] = blk_ca

    @pl.when(j != 0)
    def _():
        ca_acc[pl.ds(b, 1), :] = jnp.maximum(ca_acc[pl.ds(b, 1), :], blk_ca)

    @pl.when(jnp.logical_and(b == B - 1, j == N_BLOCKS - 1))
    def _():
        ca_ref[...] = ca_acc[...]


def _pn_normalize(h, stats, gamma, beta, mask4):
    gb = jnp.stack([gamma, beta], axis=0)
    h2, papre4, ca = pl.pallas_call(
        _norm_body,
        grid=(B, N_BLOCKS),
        in_specs=[
            pl.BlockSpec((1, BN_ROWS, 64), lambda b, j: (b, j, 0)),
            pl.BlockSpec((2, 64), lambda b, j: (0, 0)),
            pl.BlockSpec((2, 64), lambda b, j: (0, 0)),
            pl.BlockSpec((1, 1, 1, BN_ROWS), lambda b, j: (b, j, 0, 0)),
        ],
        out_specs=[
            pl.BlockSpec((1, BN_ROWS, 64), lambda b, j: (b, j, 0)),
            pl.BlockSpec((1, 1, 1, BN_ROWS), lambda b, j: (b, j, 0, 0)),
            pl.BlockSpec((B, 64), lambda b, j: (0, 0)),
        ],
        out_shape=[
            jax.ShapeDtypeStruct((B, NP_PAD, 64), jnp.float32),
            jax.ShapeDtypeStruct((B, N_BLOCKS, 1, BN_ROWS), jnp.float32),
            jax.ShapeDtypeStruct((B, 64), jnp.float32),
        ],
        scratch_shapes=[pltpu.VMEM((B, 64), jnp.float32)],
    )(h, stats, gb, mask4)
    return h2, papre4.reshape(B, NP_PAD), ca


def _attn_body(papre_ref, capre_ref, w1t_ref, b1_ref, w2t_ref, b2_ref,
               cw1t_ref, cb1_ref, cw2t_ref, cb2_ref, pa_ref, ca_ref):
    t = jnp.dot(papre_ref[...], w1t_ref[...], preferred_element_type=jnp.float32)
    t = jnp.maximum(t + b1_ref[0, :][None, :], 0.0)
    pa_ref[...] = jnp.dot(t, w2t_ref[...], preferred_element_type=jnp.float32) \
        + b2_ref[0, :][None, :]
    c = jnp.dot(capre_ref[...], cw1t_ref[...], preferred_element_type=jnp.float32)
    c = jnp.maximum(c + cb1_ref[0, :][None, :], 0.0)
    ca_ref[...] = jnp.dot(c, cw2t_ref[...], preferred_element_type=jnp.float32) \
        + cb2_ref[0, :][None, :]


def _pn_attention(papre, capre, p):
    w1t = _pad_to(p['pa_w1'], NP_PAD, 1).T
    w2t = _pad_to(p['pa_w2'], NP_PAD, 0).T
    b1 = p['pa_b1'][None, :]
    b2 = _pad_to(p['pa_b2'], NP_PAD, 0)[None, :]
    cw1t = p['ca_w1'].T
    cw2t = p['ca_w2'].T
    cb1 = p['ca_b1'][None, :]
    cb2 = p['ca_b2'][None, :]
    w1tp = _pad_to(w1t, 8, 1)
    w2tp = _pad_to(w2t, 8, 0)
    b1p = _pad_to(b1, 8, 1)
    cw1tp = _pad_to(cw1t, 8, 1)
    cw2tp = _pad_to(cw2t, 8, 0)
    cb1p = _pad_to(cb1, 8, 1)
    pa, ca = pl.pallas_call(
        _attn_body,
        out_shape=[
            jax.ShapeDtypeStruct((B, NP_PAD), jnp.float32),
            jax.ShapeDtypeStruct((B, 64), jnp.float32),
        ],
    )(papre, capre, w1tp, b1p, w2tp, b2, cw1tp, cb1p, cw2tp, cb2)
    return pa, ca


def _combine_body(h2_ref, pa_ref, ca_ref, out_ref):
    b = pl.program_id(0)
    pa = pa_ref[0, 0, 0]
    ca = ca_ref[pl.ds(b, 1), :]
    w = jax.nn.sigmoid(pa[:, None] * ca)
    out_ref[0] = h2_ref[0] * w


def _pn_combine(h2, pa, ca):
    pa4 = pa.reshape(B, N_BLOCKS, 1, BN_ROWS)
    return pl.pallas_call(
        _combine_body,
        grid=(B, N_BLOCKS),
        in_specs=[
            pl.BlockSpec((1, BN_ROWS, 64), lambda b, j: (b, j, 0)),
            pl.BlockSpec((1, 1, 1, BN_ROWS), lambda b, j: (b, j, 0, 0)),
            pl.BlockSpec((B, 64), lambda b, j: (0, 0)),
        ],
        out_specs=pl.BlockSpec((1, BN_ROWS, 64), lambda b, j: (b, j, 0)),
        out_shape=jax.ShapeDtypeStruct((B, NP_PAD, 64), jnp.float32),
    )(h2, pa4, ca)


def _pointnet(x_padded, mask4, p):
    cp = x_padded.shape[-1]
    wt = _pad_to(p['W'].T, cp, 0)
    h, stats = _pn_matmul_stats(x_padded, wt)
    h2, papre, capre = _pn_normalize(h, stats, p['gamma'], p['beta'], mask4)
    pa, ca = _pn_attention(papre, capre, p)
    return _pn_combine(h2, pa, ca)


def _conv_stats_body(v_ref, wt_ref, e_ref, ks_ref, vc_ref, stats_ref, acc_ref,
                     *, nv):
    b = pl.program_id(0)
    j = pl.program_id(1)

    @pl.when(jnp.logical_and(b == 0, j == 0))
    def _():
        acc_ref[...] = jnp.zeros_like(acc_ref)

    vc = jnp.dot(v_ref[0], wt_ref[...], preferred_element_type=jnp.float32)
    vc_ref[0] = vc
    m = (e_ref[0, 0, 0] * (ks_ref[0, 0, 0] < nv).astype(jnp.float32))[:, None]
    vcm = vc * m
    acc_ref[0, :] += jnp.sum(vcm, axis=0)
    acc_ref[1, :] += jnp.sum(vcm * vc, axis=0)

    @pl.when(jnp.logical_and(b == B - 1, j == N_BLOCKS - 1))
    def _():
        stats_ref[...] = acc_ref[...]


def _conv_norm_body(vc_ref, stats_ref, gb_ref, y_ref, yemp_ref, *, nv):
    b = pl.program_id(0)
    j = pl.program_id(1)
    cnt = float(B * nv)
    mu = stats_ref[0, :] / cnt
    var = stats_ref[1, :] / cnt - mu * mu
    rs = jax.lax.rsqrt(var + 1e-3)
    g = gb_ref[0, :]
    be = gb_ref[1, :]
    y_ref[0] = jnp.maximum((vc_ref[0] - mu[None, :]) * rs[None, :]
                           * g[None, :] + be[None, :], 0.0)

    @pl.when(jnp.logical_and(b == B - 1, j == N_BLOCKS - 1))
    def _():
        yemp_ref[...] = jnp.maximum((0.0 - mu[None, :]) * rs[None, :]
                                    * g[None, :] + be[None, :], 0.0)


def _conv_bn(pool, conv_w, bn_g, bn_b, e4, ks4, nv):
    wt = conv_w.T
    vc, stats = pl.pallas_call(
        functools.partial(_conv_stats_body, nv=nv),
        grid=(B, N_BLOCKS),
        in_specs=[
            pl.BlockSpec((1, BN_ROWS, 64), lambda b, j: (b, j, 0)),
            pl.BlockSpec((64, 64), lambda b, j: (0, 0)),
            pl.BlockSpec((1, 1, 1, BN_ROWS), lambda b, j: (b, j, 0, 0)),
            pl.BlockSpec((1, 1, 1, BN_ROWS), lambda b, j: (b, j, 0, 0)),
        ],
        out_specs=[
            pl.BlockSpec((1, BN_ROWS, 64), lambda b, j: (b, j, 0)),
            pl.BlockSpec((2, 64), lambda b, j: (0, 0)),
        ],
        out_shape=[
            jax.ShapeDtypeStruct((B, NP_PAD, 64), jnp.float32),
            jax.ShapeDtypeStruct((2, 64), jnp.float32),
        ],
        scratch_shapes=[pltpu.VMEM((2, 64), jnp.float32)],
    )(pool, wt, e4, ks4)
    gb = jnp.stack([bn_g, bn_b], axis=0)
    y, yemp = pl.pallas_call(
        functools.partial(_conv_norm_body, nv=nv),
        grid=(B, N_BLOCKS),
        in_specs=[
            pl.BlockSpec((1, BN_ROWS, 64), lambda b, j: (b, j, 0)),
            pl.BlockSpec((2, 64), lambda b, j: (0, 0)),
            pl.BlockSpec((2, 64), lambda b, j: (0, 0)),
        ],
        out_specs=[
            pl.BlockSpec((1, BN_ROWS, 64), lambda b, j: (b, j, 0)),
            pl.BlockSpec((1, 64), lambda b, j: (0, 0)),
        ],
        out_shape=[
            jax.ShapeDtypeStruct((B, NP_PAD, 64), jnp.float32),
            jax.ShapeDtypeStruct((1, 64), jnp.float32),
        ],
    )(vc, stats, gb)
    return y, yemp


def _bilerp_body(i_ref, w_ref, out_ref):
    ii = i_ref[0]
    w = w_ref[0, 0]
    out_ref[0] = (ii[:, 0, :] * w[:, 0:1] + ii[:, 1, :] * w[:, 1:2]
                  + ii[:, 2, :] * w[:, 2:3] + ii[:, 3, :] * w[:, 3:4])


def _bilerp(ivals, w4):
    return pl.pallas_call(
        _bilerp_body,
        grid=(B, N_BLOCKS),
        in_specs=[
            pl.BlockSpec((1, BN_ROWS, 4, 64), lambda b, j: (b, j, 0, 0)),
            pl.BlockSpec((1, 1, BN_ROWS, 4), lambda b, j: (b, j, 0, 0)),
        ],
        out_specs=pl.BlockSpec((1, BN_ROWS, 64), lambda b, j: (b, j, 0)),
        out_shape=jax.ShapeDtypeStruct((B, NP_PAD, 64), jnp.float32),
    )(ivals, w4)


def _round_up(x, m):
    return (x + m - 1) // m * m


def _view_precompute(p_planes, mask_pl, grid, ranges):
    gx, gy = grid
    nv = gx * gy
    s_stride = nv + 2
    s_len = _round_up(B * s_stride, NS * 16 * CH)
    key_pl, valid_pl, cen_pl, pay_pl, idxg_pl, ids4_pl, w4_pl = _prep(
        p_planes, mask_pl, grid, ranges)
    key = key_pl.reshape(B, NP_PAD)
    iota = jnp.broadcast_to(jnp.arange(NP_PAD, dtype=jnp.int32), (B, NP_PAD))
    ks, perm = jax.vmap(lambda k, i: lax.sort((k, i), num_keys=1))(key, iota)
    kp = jnp.concatenate([jnp.full((B, 1), -1, jnp.int32), ks[:, :-1]], 1)
    kn = jnp.concatenate([ks[:, 1:], jnp.full((B, 1), -1, jnp.int32)], 1)
    ks4 = ks.reshape(B, N_BLOCKS, 1, BN_ROWS)
    kp4 = kp.reshape(B, N_BLOCKS, 1, BN_ROWS)
    kn4 = kn.reshape(B, N_BLOCKS, 1, BN_ROWS)
    permg = (perm + (jnp.arange(B, dtype=jnp.int32) * NP_PAD)[:, None]
             ).reshape(M_PTS)
    pay = pay_pl.transpose(0, 1, 3, 4, 2).reshape(M_PTS, 16)
    pay_sorted = _sc_gather_rows(pay, permg)
    tot, e4, sval4, scell4 = _seg_scan(ks4, kp4, kn4,
                                       pay_sorted.reshape(B, NP_PAD, 16),
                                       is_max=False, with_slots=True,
                                       s_stride=s_stride, nv=nv)
    s_map = _sc_scatter_add(scell4.reshape(M_PTS), sval4.reshape(M_PTS),
                            s_len)
    stats_table = jnp.concatenate(
        [jnp.zeros((N_EMPTY, 16), jnp.float32), tot.reshape(M_PTS, 16)],
        axis=0)
    gstats = _sc_gather2(s_map, stats_table, idxg_pl.reshape(M_PTS))
    g_planes = gstats.reshape(B, N_BLOCKS, R24, 128, 16).transpose(
        0, 1, 4, 2, 3)
    pview_pl = _stats_finalize(g_planes, p_planes, valid_pl, mask_pl, cen_pl)
    ids4 = ids4_pl.transpose(0, 1, 3, 4, 2).reshape(M_PTS * 4)
    w4 = w4_pl.transpose(0, 1, 3, 4, 2).reshape(B, N_BLOCKS, BN_ROWS, 4)
    return dict(nv=nv, s_stride=s_stride, s_map=s_map, ks4=ks4, kp4=kp4,
                kn4=kn4, e4=e4, permg=permg, ids4=ids4, w4=w4,
                pview_pl=pview_pl)


def _single_view(x_pad, mask4, vc, grid, params):
    nv = vc['nv']
    h = _pointnet(x_pad, mask4, params['pn'])
    h_sorted = _sc_gather_rows(h.reshape(M_PTS, 64), vc['permg'])
    pool = _seg_scan(vc['ks4'], vc['kp4'], vc['kn4'],
                     h_sorted.reshape(B, NP_PAD, 64), is_max=True,
                     with_slots=False)
    y, yemp = _conv_bn(pool, params['conv_w'], params['bn_g'],
                       params['bn_b'], vc['e4'], vc['ks4'], nv)
    img_table = jnp.concatenate(
        [jnp.broadcast_to(yemp, (N_EMPTY, 64)), y.reshape(M_PTS, 64)], axis=0)
    ivals = _sc_gather2(vc['s_map'], img_table, vc['ids4'])
    return _bilerp(ivals.reshape(B, NP_PAD, 4, 64), vc['w4'])


def _to_planes(x_pad):
    return x_pad.reshape(B, N_BLOCKS, R24, 128, x_pad.shape[-1]).transpose(
        0, 1, 4, 2, 3)


def kernel(points_xyz, points_feature, points_mask, params):
    p_pad = _pad_pts(points_xyz)
    mask_pad = _pad_pts(points_mask)
    mask4 = mask_pad.reshape(B, N_BLOCKS, 1, BN_ROWS)
    maskp = _pad_pts(points_mask, value=-1.0)
    mask_pl = maskp.reshape(B, N_BLOCKS, R24, 128)
    x = points_xyz
    rho = jnp.sqrt(x[..., 0] ** 2 + x[..., 1] ** 2)
    theta = jnp.arctan2(x[..., 1], x[..., 0])
    pc = jnp.stack([theta, x[..., 2], rho], axis=-1)
    pc_pad = _pad_pts(pc)
    p_planes = _to_planes(p_pad)
    pc_planes = _to_planes(pc_pad)

    xyc = _view_precompute(p_planes, mask_pl, XY_GRID, XY_R)
    cyc = _view_precompute(pc_planes, mask_pl, CYL_GRID, CYL_R)

    feat_pl = jnp.concatenate(
        [p_planes, xyc['pview_pl'], pc_planes, cyc['pview_pl'],
         _pad_pts(points_feature).reshape(B, N_BLOCKS, 1, R24, 128)], axis=2)
    feat = feat_pl.transpose(0, 1, 3, 4, 2).reshape(B, NP_PAD, 45)
    feat = _pad_to(feat, 128, 2)
    x1 = _pointnet(feat, mask4, params['pn1'])
    x1p = _pad_to(x1, 128, 2)
    xxy = _single_view(x1p, mask4, xyc, XY_GRID, params['xy'])
    xcy = _single_view(x1p, mask4, cyc, CYL_GRID, params['cyl'])
    xpw = _pointnet(x1p, mask4, params['pn2'])
    x2 = jnp.concatenate([xxy, xcy, xpw], axis=-1)
    x2 = _pad_to(x2, 256, 2)
    x3 = _pointnet(x2, mask4, params['pn3'])

    x3_sorted = _sc_gather_rows(x3.reshape(M_PTS, 64), xyc['permg'])
    pool = _seg_scan(xyc['ks4'], xyc['kp4'], xyc['kn4'],
                     x3_sorted.reshape(B, NP_PAD, 64), is_max=True,
                     with_slots=False)
    nv = xyc['nv']
    ks = xyc['ks4'].reshape(B, NP_PAD)
    e = xyc['e4'].reshape(B, NP_PAD) > 0.5
    real = e & (ks < nv)
    boff = (jnp.arange(B, dtype=jnp.int32) * nv)[:, None]
    cell = jnp.where(real, ks + boff, B * nv).reshape(M_PTS)
    grid_flat = jnp.zeros((B * nv, 64), jnp.float32)
    grid_flat = grid_flat.at[cell].set(pool.reshape(M_PTS, 64),
                                       mode='drop',
                                       unique_indices=True)
    pil = grid_flat.reshape(B, XY_GRID[0], XY_GRID[1], 64)
    return pil.transpose(0, 3, 2, 1)


# speedup vs baseline: 1.0054x; 1.0054x over previous
import functools
import jax
import jax.numpy as jnp
from jax import lax
from jax.experimental import pallas as pl
from jax.experimental.pallas import tpu as pltpu
from jax.experimental.pallas import tpu_sc as plsc

XY_GRID = (432, 496)
XY_R = ((0.0, 69.12), (-39.68, 39.68), (-3.0, 1.0))
CYL_GRID = (2560, 100)
CYL_R = ((-3.141592653589793, 3.141592653589793), (-3.0, 1.0), (0.0, 69.12))
NPTS = 30000
B = 2

NP_PAD = 30720
BN_ROWS = 3072
N_BLOCKS = NP_PAD // BN_ROWS
M_PTS = B * NP_PAD

try:
    _SC = plsc.get_sparse_core_info()
    NC, NS, L = _SC.num_cores, _SC.num_subcores, _SC.num_lanes
except Exception:
    NC, NS, L = 2, 16, 16
NW = NC * NS
CH = 128
_SC_PARAMS = pltpu.CompilerParams(use_tc_tiling_on_sc=False)


def _pad_pts(x, value=0.0):
    pad = [(0, 0)] * x.ndim
    pad[1] = (0, NP_PAD - x.shape[1])
    return jnp.pad(x, pad, constant_values=value)


def _pad_to(x, n, axis, value=0.0):
    pad = [(0, 0)] * x.ndim
    pad[axis] = (0, n - x.shape[axis])
    return jnp.pad(x, pad, constant_values=value)


KQ = 1


@functools.lru_cache(maxsize=None)
def _sc_gather_rows_fn(t_rows, d, m):
    b_per_w = m // NW
    n_inner = b_per_w // (KQ * CH)
    mesh = plsc.VectorSubcoreMesh(core_axis_name="c", subcore_axis_name="s")

    @functools.partial(
        pl.kernel, mesh=mesh,
        out_type=jax.ShapeDtypeStruct((m, d), jnp.float32),
        compiler_params=_SC_PARAMS,
        scratch_types=[
            pltpu.VMEM((KQ, CH), jnp.int32),
            pltpu.VMEM((KQ * CH, d), jnp.float32),
            pltpu.SemaphoreType.DMA,
        ],
    )
    def k(table_hbm, idx_hbm, out_hbm, idx_v, rows_v, sem):
        wid = lax.axis_index("s") * NC + lax.axis_index("c")
        base = wid * b_per_w

        def body(g, _):
            off = base + g * (KQ * CH)
            pltpu.sync_copy(idx_hbm.at[pl.ds(off // CH, KQ)], idx_v)
            cps = [pltpu.async_copy(table_hbm.at[idx_v.at[q]],
                                    rows_v.at[pl.ds(q * CH, CH)], sem)
                   for q in range(KQ)]
            for cp in cps:
                cp.wait()
            pltpu.sync_copy(rows_v, out_hbm.at[pl.ds(off, KQ * CH)])
            return 0

        lax.fori_loop(0, n_inner, body, 0)

    return k


def _sc_gather_rows(table, idx):
    m = idx.shape[0]
    return _sc_gather_rows_fn(table.shape[0], table.shape[1], m)(
        table, idx.reshape(m // CH, CH))


@functools.lru_cache(maxsize=None)
def _sc_gather2_fn(s_len, t_rows, d, m):
    b_per_w = m // NW
    n_inner = b_per_w // (KQ * CH)
    mesh = plsc.VectorSubcoreMesh(core_axis_name="c", subcore_axis_name="s")

    @functools.partial(
        pl.kernel, mesh=mesh,
        out_type=jax.ShapeDtypeStruct((m, d), jnp.float32),
        compiler_params=_SC_PARAMS,
        scratch_types=[
            pltpu.VMEM((KQ, CH), jnp.int32),
            pltpu.VMEM((KQ, CH), jnp.int32),
            pltpu.VMEM((KQ * CH, d), jnp.float32),
            pltpu.SemaphoreType.DMA,
        ],
    )
    def k(s_hbm, table_hbm, idx_hbm, out_hbm, idx_v, slot_v, rows_v, sem):
        wid = lax.axis_index("s") * NC + lax.axis_index("c")
        base = wid * b_per_w

        def body(g, _):
            off = base + g * (KQ * CH)
            pltpu.sync_copy(idx_hbm.at[pl.ds(off // CH, KQ)], idx_v)
            cps = [pltpu.async_copy(s_hbm.at[idx_v.at[q]], slot_v.at[q], sem)
                   for q in range(KQ)]
            for cp in cps:
                cp.wait()
            cps = [pltpu.async_copy(table_hbm.at[slot_v.at[q]],
                                    rows_v.at[pl.ds(q * CH, CH)], sem)
                   for q in range(KQ)]
            for cp in cps:
                cp.wait()
            pltpu.sync_copy(rows_v, out_hbm.at[pl.ds(off, KQ * CH)])
            return 0

        lax.fori_loop(0, n_inner, body, 0)

    return k


def _sc_gather2(s_map, table, idx):
    m = idx.shape[0]
    return _sc_gather2_fn(s_map.shape[0], table.shape[0], table.shape[1],
                          m)(s_map, table, idx.reshape(m // CH, CH))


@functools.lru_cache(maxsize=None)
def _sc_scatter_add_fn(s_len, m):
    half = m // NC
    b_per_w = half // NS
    n_inner = b_per_w // CH
    tile_len = s_len // NS
    n_ms = tile_len // (16 * CH)
    mesh = plsc.VectorSubcoreMesh(core_axis_name="c", subcore_axis_name="s")

    @functools.partial(
        pl.kernel, mesh=mesh,
        out_type=jax.ShapeDtypeStruct((NC, s_len), jnp.int32),
        compiler_params=_SC_PARAMS,
        scratch_types=[
            pltpu.VMEM_SHARED((s_len,), jnp.int32),
            pltpu.VMEM((16 * CH,), jnp.int32),
            pltpu.VMEM((CH,), jnp.int32),
            pltpu.VMEM((CH,), jnp.int32),
        ],
    )
    def k(idx_hbm, val_hbm, out_hbm, smem, zbuf, idx_v, val_v):
        c = lax.axis_index("c")
        s = lax.axis_index("s")

        def zb(i, _):
            zbuf[pl.ds(i * L, L)] = jnp.zeros((L,), jnp.int32)
            return 0

        lax.fori_loop(0, (16 * CH) // L, zb, 0)
        tbase = s * tile_len

        def ms(i, _):
            pltpu.sync_copy(zbuf, smem.at[pl.ds(tbase + i * 16 * CH, 16 * CH)])
            return 0

        lax.fori_loop(0, n_ms, ms, 0)
        plsc.subcore_barrier()

        base = c * half + s * b_per_w

        def body(g, _):
            off = base + g * CH
            pltpu.sync_copy(idx_hbm.at[pl.ds(off, CH)], idx_v)
            pltpu.sync_copy(val_hbm.at[pl.ds(off, CH)], val_v)
            pltpu.sync_copy(val_v, smem.at[idx_v], add=True)
            return 0

        lax.fori_loop(0, n_inner, body, 0)
        plsc.subcore_barrier()

        def wr(i, _):
            pltpu.sync_copy(smem.at[pl.ds(tbase + i * 16 * CH, 16 * CH)],
                            zbuf)
            pltpu.sync_copy(zbuf, out_hbm.at[c, pl.ds(tbase + i * 16 * CH,
                                                      16 * CH)])
            return 0

        lax.fori_loop(0, n_ms, wr, 0)

    return k


N_EMPTY = 64


def _sc_scatter_add(idx, vals, s_len):
    parts = _sc_scatter_add_fn(s_len, idx.shape[0])(idx, vals)
    s_raw = parts[0] + parts[1]
    spread = jnp.arange(s_len, dtype=jnp.int32) & (N_EMPTY - 1)
    return jnp.where(s_raw == 0, spread, s_raw - 1 + N_EMPTY)


def _shift_down(x, s, fill):
    top = jnp.full((s,) + x.shape[1:], fill, x.dtype)
    return jnp.concatenate([top, x[: x.shape[0] - s]], axis=0)


def _seg_scan_body(is_max, c, ks_ref, kp_ref, kn_ref, v_ref, tot_ref, e_ref,
                   sval_ref, scell_ref, carry_ref, *, s_stride, nv):
    b = pl.program_id(0)
    j = pl.program_id(1)
    ks = ks_ref[0, 0, 0][:, None]
    kp = kp_ref[0, 0, 0][:, None]
    ident = jnp.float32(-1e30) if is_max else jnp.float32(0.0)
    vals = v_ref[0]
    n = vals.shape[0]
    s = 1
    while s < n:
        ks_s = _shift_down(ks, s, -2)
        v_s = _shift_down(vals, s, ident)
        gate = ks == ks_s
        if is_max:
            vals = jnp.maximum(vals, jnp.where(gate, v_s, ident))
        else:
            vals = vals + jnp.where(gate, v_s, 0.0)
        s *= 2

    @pl.when(j == 0)
    def _():
        carry_ref[...] = jnp.full_like(carry_ref, ident)

    kp_first = kp_ref[0, 0, 0, 0]
    span = ks == kp_first
    carry = carry_ref[0, :][None, :]
    if is_max:
        vals = jnp.where(span, jnp.maximum(vals, carry), vals)
    else:
        vals = jnp.where(span, vals + carry, vals)
    carry_ref[0, :] = vals[n - 1, :]
    tot_ref[0] = vals
    if e_ref is not None:
        kn = kn_ref[0, 0, 0]
        e = ks_ref[0, 0, 0] != kn
        rowg = b * NP_PAD + j * n + lax.broadcasted_iota(jnp.int32, (n,), 0)
        e_ref[0, 0, 0] = e.astype(jnp.float32)
        sval_ref[0, 0, 0] = jnp.where(e, rowg + 1, 0)
        scell_ref[0, 0, 0] = b * s_stride + ks_ref[0, 0, 0]


def _seg_scan(ks4, kp4, kn4, vals, is_max, with_slots, s_stride=0, nv=0):
    c = vals.shape[-1]
    outs = [jax.ShapeDtypeStruct((B, NP_PAD, c), jnp.float32)]
    out_specs = [pl.BlockSpec((1, BN_ROWS, c), lambda b, j: (b, j, 0))]
    if with_slots:
        for dt in (jnp.float32, jnp.int32, jnp.int32):
            outs.append(jax.ShapeDtypeStruct((B, N_BLOCKS, 1, BN_ROWS), dt))
            out_specs.append(
                pl.BlockSpec((1, 1, 1, BN_ROWS), lambda b, j: (b, j, 0, 0)))
        body = functools.partial(_seg_scan_body, is_max, c,
                                 s_stride=s_stride, nv=nv)
    else:
        def body(ks_ref, kp_ref, kn_ref, v_ref, tot_ref, carry_ref):
            _seg_scan_body(is_max, c, ks_ref, kp_ref, kn_ref, v_ref, tot_ref,
                           None, None, None, carry_ref, s_stride=0, nv=0)
    res = pl.pallas_call(
        body,
        grid=(B, N_BLOCKS),
        in_specs=[
            pl.BlockSpec((1, 1, 1, BN_ROWS), lambda b, j: (b, j, 0, 0)),
            pl.BlockSpec((1, 1, 1, BN_ROWS), lambda b, j: (b, j, 0, 0)),
            pl.BlockSpec((1, 1, 1, BN_ROWS), lambda b, j: (b, j, 0, 0)),
            pl.BlockSpec((1, BN_ROWS, c), lambda b, j: (b, j, 0)),
        ],
        out_specs=out_specs if with_slots else out_specs[0],
        out_shape=outs if with_slots else outs[0],
        scratch_shapes=[pltpu.VMEM((1, c), jnp.float32)],
    )(ks4, kp4, kn4, vals)
    return res


R24 = BN_ROWS // 128


def _prep_body(p_ref, mask_ref, key_ref, valid_ref, cen_ref, pay_ref,
               idxg_ref, ids4_ref, w4_ref, *, grid, ranges, s_stride):
    gx, gy = grid
    (x0, x1), (y0, y1), (z0, z1) = ranges
    b = pl.program_id(0)
    vsx = (x1 - x0) / gx
    vsy = (y1 - y0) / gy
    vsz = (z1 - z0) / 1.0
    px = p_ref[0, 0, 0]
    py = p_ref[0, 0, 1]
    pz = p_ref[0, 0, 2]
    mask = mask_ref[0, 0]
    fx = (px - x0) / vsx
    fy = (py - y0) / vsy
    fz = (pz - z0) / vsz
    cx = jnp.floor(fx)
    cy = jnp.floor(fy)
    cz = jnp.floor(fz)
    inr = ((cx >= 0) & (cx < gx) & (cy >= 0) & (cy < gy)
           & (cz >= 0) & (cz < 1))
    valid = inr & (mask > 0.5)
    ccx = jnp.clip(cx, 0, gx - 1)
    ccy = jnp.clip(cy, 0, gy - 1)
    idx = (ccx * gy + ccy).astype(jnp.int32)
    nv = gx * gy
    key = jnp.where(valid, idx, nv)
    key = jnp.where(mask > -0.5, key, nv + 1)
    key_ref[0, 0] = key
    valid_ref[0, 0] = valid.astype(jnp.float32)
    cen_ref[0, 0, 0] = (ccx + 0.5) * vsx + x0
    cen_ref[0, 0, 1] = (ccy + 0.5) * vsy + y0
    cen_ref[0, 0, 2] = jnp.full_like(px, 0.5 * vsz + z0)
    one = jnp.ones_like(px)
    zero = jnp.zeros_like(px)
    for c, v in enumerate([one, px, py, pz, px * px, py * py, pz * pz,
                           px * py, px * pz, py * pz,
                           fx, fy, zero, zero, zero, zero]):
        pay_ref[0, 0, c] = v
    idxg_ref[0, 0] = b * s_stride + idx
    xq0 = jnp.floor(fx)
    yq0 = jnp.floor(fy)
    x0c = jnp.clip(xq0, 0, gx - 1).astype(jnp.int32)
    x1c = jnp.clip(xq0 + 1, 0, gx - 1).astype(jnp.int32)
    y0c = jnp.clip(yq0, 0, gy - 1).astype(jnp.int32)
    y1c = jnp.clip(yq0 + 1, 0, gy - 1).astype(jnp.int32)
    base = b * s_stride
    ids4_ref[0, 0, 0] = base + x0c * gy + y0c
    ids4_ref[0, 0, 1] = base + x0c * gy + y1c
    ids4_ref[0, 0, 2] = base + x1c * gy + y0c
    ids4_ref[0, 0, 3] = base + x1c * gy + y1c
    wxa = (xq0 + 1.0) - fx
    wxc = fx - xq0
    wya = (yq0 + 1.0) - fy
    wyb = fy - yq0
    realrow = (mask > -0.5).astype(jnp.float32)
    w4_ref[0, 0, 0] = wxa * wya * realrow
    w4_ref[0, 0, 1] = wxa * wyb * realrow
    w4_ref[0, 0, 2] = wxc * wya * realrow
    w4_ref[0, 0, 3] = wxc * wyb * realrow


def _prep(p_planes, mask_planes, grid, ranges):
    gx, gy = grid
    s_stride = gx * gy + 2
    body = functools.partial(_prep_body, grid=grid, ranges=ranges,
                             s_stride=s_stride)

    def pm(b, j):
        return (b, j, 0, 0)

    def pm5(b, j):
        return (b, j, 0, 0, 0)

    return pl.pallas_call(
        body,
        grid=(B, N_BLOCKS),
        in_specs=[
            pl.BlockSpec((1, 1, 3, R24, 128), pm5),
            pl.BlockSpec((1, 1, R24, 128), pm),
        ],
        out_specs=[
            pl.BlockSpec((1, 1, R24, 128), pm),
            pl.BlockSpec((1, 1, R24, 128), pm),
            pl.BlockSpec((1, 1, 3, R24, 128), pm5),
            pl.BlockSpec((1, 1, 16, R24, 128), pm5),
            pl.BlockSpec((1, 1, R24, 128), pm),
            pl.BlockSpec((1, 1, 4, R24, 128), pm5),
            pl.BlockSpec((1, 1, 4, R24, 128), pm5),
        ],
        out_shape=[
            jax.ShapeDtypeStruct((B, N_BLOCKS, R24, 128), jnp.int32),
            jax.ShapeDtypeStruct((B, N_BLOCKS, R24, 128), jnp.float32),
            jax.ShapeDtypeStruct((B, N_BLOCKS, 3, R24, 128), jnp.float32),
            jax.ShapeDtypeStruct((B, N_BLOCKS, 16, R24, 128), jnp.float32),
            jax.ShapeDtypeStruct((B, N_BLOCKS, R24, 128), jnp.int32),
            jax.ShapeDtypeStruct((B, N_BLOCKS, 4, R24, 128), jnp.int32),
            jax.ShapeDtypeStruct((B, N_BLOCKS, 4, R24, 128), jnp.float32),
        ],
    )(p_planes, mask_planes)


def _stats_finalize_body(g_ref, p_ref, valid_ref, rm_ref, cen_ref, out_ref):
    m = valid_ref[0, 0]
    realrow = (rm_ref[0, 0] > -0.5).astype(jnp.float32)
    mr = m * realrow
    cnt = g_ref[0, 0, 0]
    cntc = jnp.maximum(cnt, 1.0)
    rc = 1.0 / cntc
    p3 = [p_ref[0, 0, c] for c in range(3)]
    mean = [g_ref[0, 0, 1 + c] * rc for c in range(3)]
    for c in range(3):
        out_ref[0, 0, c] = (p3[c] - cen_ref[0, 0, c]) * realrow
    out_ref[0, 0, 3] = cnt * mr
    for c in range(3):
        out_ref[0, 0, 4 + c] = (p3[c] - mean[c]) * mr
    e2 = [g_ref[0, 0, 4 + c] * rc for c in range(6)]
    cov = [e2[0] - mean[0] * mean[0], e2[3] - mean[0] * mean[1],
           e2[4] - mean[0] * mean[2],
           e2[3] - mean[0] * mean[1], e2[1] - mean[1] * mean[1],
           e2[5] - mean[1] * mean[2],
           e2[4] - mean[0] * mean[2], e2[5] - mean[1] * mean[2],
           e2[2] - mean[2] * mean[2]]
    for c in range(9):
        out_ref[0, 0, 7 + c] = cov[c] * mr
    for c in range(3):
        out_ref[0, 0, 16 + c] = mean[c] * mr


def _stats_finalize(g_planes, p_planes, valid_pl, rm_pl, cen_planes):
    def pm(b, j):
        return (b, j, 0, 0)

    def pm5(b, j):
        return (b, j, 0, 0, 0)

    return pl.pallas_call(
        _stats_finalize_body,
        grid=(B, N_BLOCKS),
        in_specs=[
            pl.BlockSpec((1, 1, 16, R24, 128), pm5),
            pl.BlockSpec((1, 1, 3, R24, 128), pm5),
            pl.BlockSpec((1, 1, R24, 128), pm),
            pl.BlockSpec((1, 1, R24, 128), pm),
            pl.BlockSpec((1, 1, 3, R24, 128), pm5),
        ],
        out_specs=pl.BlockSpec((1, 1, 19, R24, 128), pm5),
        out_shape=jax.ShapeDtypeStruct((B, N_BLOCKS, 19, R24, 128),
                                       jnp.float32),
    )(g_planes, p_planes, valid_pl, rm_pl, cen_planes)


def _mm_stats_body(x_ref, wt_ref, h_ref, stats_ref, acc_ref):
    b = pl.program_id(0)
    j = pl.program_id(1)

    @pl.when(jnp.logical_and(b == 0, j == 0))
    def _():
        acc_ref[...] = jnp.zeros_like(acc_ref)

    h = jnp.dot(x_ref[0], wt_ref[...], preferred_element_type=jnp.float32)
    h_ref[0] = h
    acc_ref[0, :] += jnp.sum(h, axis=0)
    acc_ref[1, :] += jnp.sum(h * h, axis=0)

    @pl.when(jnp.logical_and(b == B - 1, j == N_BLOCKS - 1))
    def _():
        stats_ref[...] = acc_ref[...]


def _pn_matmul_stats(x, wt):
    cp = x.shape[-1]
    return pl.pallas_call(
        _mm_stats_body,
        grid=(B, N_BLOCKS),
        in_specs=[
            pl.BlockSpec((1, BN_ROWS, cp), lambda b, j: (b, j, 0)),
            pl.BlockSpec((cp, 64), lambda b, j: (0, 0)),
        ],
        out_specs=[
            pl.BlockSpec((1, BN_ROWS, 64), lambda b, j: (b, j, 0)),
            pl.BlockSpec((2, 64), lambda b, j: (0, 0)),
        ],
        out_shape=[
            jax.ShapeDtypeStruct((B, NP_PAD, 64), jnp.float32),
            jax.ShapeDtypeStruct((2, 64), jnp.float32),
        ],
        scratch_shapes=[pltpu.VMEM((2, 64), jnp.float32)],
    )(x, wt)


def _norm_body(h_ref, stats_ref, gb_ref, mask_ref, h2_ref, papre_ref, ca_ref,
               ca_acc):
    b = pl.program_id(0)
    j = pl.program_id(1)
    cnt = float(B * NPTS)
    mu = stats_ref[0, :] / cnt
    var = stats_ref[1, :] / cnt - mu * mu
    gamma = gb_ref[0, :]
    beta = gb_ref[1, :]
    h = h_ref[0]
    h2 = (h - mu[None, :]) * jax.lax.rsqrt(var + 1e-3)[None, :]
    h2 = h2 * gamma[None, :] + beta[None, :]
    h2 = jnp.maximum(h2, 0.0) * mask_ref[0, 0, 0][:, None]
    h2_ref[0] = h2
    papre_ref[0, 0, 0, :] = jnp.max(h2, axis=1)
    blk_ca = jnp.max(h2, axis=0, keepdims=True)

    @pl.when(j == 0)
    def _():
        ca_acc[pl.ds(b, 1), :] = blk_ca

    @pl.when(j != 0)
    def _():
        ca_acc[pl.ds(b, 1), :] = jnp.maximum(ca_acc[pl.ds(b, 1), :], blk_ca)

    @pl.when(jnp.logical_and(b == B - 1, j == N_BLOCKS - 1))
    def _():
        ca_ref[...] = ca_acc[...]


def _pn_normalize(h, stats, gamma, beta, mask4):
    gb = jnp.stack([gamma, beta], axis=0)
    h2, papre4, ca = pl.pallas_call(
        _norm_body,
        grid=(B, N_BLOCKS),
        in_specs=[
            pl.BlockSpec((1, BN_ROWS, 64), lambda b, j: (b, j, 0)),
            pl.BlockSpec((2, 64), lambda b, j: (0, 0)),
            pl.BlockSpec((2, 64), lambda b, j: (0, 0)),
            pl.BlockSpec((1, 1, 1, BN_ROWS), lambda b, j: (b, j, 0, 0)),
        ],
        out_specs=[
            pl.BlockSpec((1, BN_ROWS, 64), lambda b, j: (b, j, 0)),
            pl.BlockSpec((1, 1, 1, BN_ROWS), lambda b, j: (b, j, 0, 0)),
            pl.BlockSpec((B, 64), lambda b, j: (0, 0)),
        ],
        out_shape=[
            jax.ShapeDtypeStruct((B, NP_PAD, 64), jnp.float32),
            jax.ShapeDtypeStruct((B, N_BLOCKS, 1, BN_ROWS), jnp.float32),
            jax.ShapeDtypeStruct((B, 64), jnp.float32),
        ],
        scratch_shapes=[pltpu.VMEM((B, 64), jnp.float32)],
    )(h, stats, gb, mask4)
    return h2, papre4.reshape(B, NP_PAD), ca


def _attn_body(papre_ref, capre_ref, w1t_ref, b1_ref, w2t_ref, b2_ref,
               cw1t_ref, cb1_ref, cw2t_ref, cb2_ref, pa_ref, ca_ref):
    t = jnp.dot(papre_ref[...], w1t_ref[...], preferred_element_type=jnp.float32)
    t = jnp.maximum(t + b1_ref[0, :][None, :], 0.0)
    pa_ref[...] = jnp.dot(t, w2t_ref[...], preferred_element_type=jnp.float32) \
        + b2_ref[0, :][None, :]
    c = jnp.dot(capre_ref[...], cw1t_ref[...], preferred_element_type=jnp.float32)
    c = jnp.maximum(c + cb1_ref[0, :][None, :], 0.0)
    ca_ref[...] = jnp.dot(c, cw2t_ref[...], preferred_element_type=jnp.float32) \
        + cb2_ref[0, :][None, :]


def _pn_attention(papre, capre, p):
    w1t = _pad_to(p['pa_w1'], NP_PAD, 1).T
    w2t = _pad_to(p['pa_w2'], NP_PAD, 0).T
    b1 = p['pa_b1'][None, :]
    b2 = _pad_to(p['pa_b2'], NP_PAD, 0)[None, :]
    cw1t = p['ca_w1'].T
    cw2t = p['ca_w2'].T
    cb1 = p['ca_b1'][None, :]
    cb2 = p['ca_b2'][None, :]
    w1tp = _pad_to(w1t, 8, 1)
    w2tp = _pad_to(w2t, 8, 0)
    b1p = _pad_to(b1, 8, 1)
    cw1tp = _pad_to(cw1t, 8, 1)
    cw2tp = _pad_to(cw2t, 8, 0)
    cb1p = _pad_to(cb1, 8, 1)
    pa, ca = pl.pallas_call(
        _attn_body,
        out_shape=[
            jax.ShapeDtypeStruct((B, NP_PAD), jnp.float32),
            jax.ShapeDtypeStruct((B, 64), jnp.float32),
        ],
    )(papre, capre, w1tp, b1p, w2tp, b2, cw1tp, cb1p, cw2tp, cb2)
    return pa, ca


def _combine_body(h2_ref, pa_ref, ca_ref, out_ref):
    b = pl.program_id(0)
    pa = pa_ref[0, 0, 0]
    ca = ca_ref[pl.ds(b, 1), :]
    w = jax.nn.sigmoid(pa[:, None] * ca)
    out_ref[0] = h2_ref[0] * w


def _pn_combine(h2, pa, ca):
    pa4 = pa.reshape(B, N_BLOCKS, 1, BN_ROWS)
    return pl.pallas_call(
        _combine_body,
        grid=(B, N_BLOCKS),
        in_specs=[
            pl.BlockSpec((1, BN_ROWS, 64), lambda b, j: (b, j, 0)),
            pl.BlockSpec((1, 1, 1, BN_ROWS), lambda b, j: (b, j, 0, 0)),
            pl.BlockSpec((B, 64), lambda b, j: (0, 0)),
        ],
        out_specs=pl.BlockSpec((1, BN_ROWS, 64), lambda b, j: (b, j, 0)),
        out_shape=jax.ShapeDtypeStruct((B, NP_PAD, 64), jnp.float32),
    )(h2, pa4, ca)


def _pointnet(x_padded, mask4, p):
    cp = x_padded.shape[-1]
    wt = _pad_to(p['W'].T, cp, 0)
    h, stats = _pn_matmul_stats(x_padded, wt)
    h2, papre, capre = _pn_normalize(h, stats, p['gamma'], p['beta'], mask4)
    pa, ca = _pn_attention(papre, capre, p)
    return _pn_combine(h2, pa, ca)


def _conv_stats_body(v_ref, wt_ref, e_ref, ks_ref, vc_ref, stats_ref, acc_ref,
                     *, nv):
    b = pl.program_id(0)
    j = pl.program_id(1)

    @pl.when(jnp.logical_and(b == 0, j == 0))
    def _():
        acc_ref[...] = jnp.zeros_like(acc_ref)

    vc = jnp.dot(v_ref[0], wt_ref[...], preferred_element_type=jnp.float32)
    vc_ref[0] = vc
    m = (e_ref[0, 0, 0] * (ks_ref[0, 0, 0] < nv).astype(jnp.float32))[:, None]
    vcm = vc * m
    acc_ref[0, :] += jnp.sum(vcm, axis=0)
    acc_ref[1, :] += jnp.sum(vcm * vc, axis=0)

    @pl.when(jnp.logical_and(b == B - 1, j == N_BLOCKS - 1))
    def _():
        stats_ref[...] = acc_ref[...]


def _conv_norm_body(vc_ref, stats_ref, gb_ref, y_ref, yemp_ref, *, nv):
    b = pl.program_id(0)
    j = pl.program_id(1)
    cnt = float(B * nv)
    mu = stats_ref[0, :] / cnt
    var = stats_ref[1, :] / cnt - mu * mu
    rs = jax.lax.rsqrt(var + 1e-3)
    g = gb_ref[0, :]
    be = gb_ref[1, :]
    y_ref[0] = jnp.maximum((vc_ref[0] - mu[None, :]) * rs[None, :]
                           * g[None, :] + be[None, :], 0.0)

    @pl.when(jnp.logical_and(b == B - 1, j == N_BLOCKS - 1))
    def _():
        yemp_ref[...] = jnp.maximum((0.0 - mu[None, :]) * rs[None, :]
                                    * g[None, :] + be[None, :], 0.0)


def _conv_bn(pool, conv_w, bn_g, bn_b, e4, ks4, nv):
    wt = conv_w.T
    vc, stats = pl.pallas_call(
        functools.partial(_conv_stats_body, nv=nv),
        grid=(B, N_BLOCKS),
        in_specs=[
            pl.BlockSpec((1, BN_ROWS, 64), lambda b, j: (b, j, 0)),
            pl.BlockSpec((64, 64), lambda b, j: (0, 0)),
            pl.BlockSpec((1, 1, 1, BN_ROWS), lambda b, j: (b, j, 0, 0)),
            pl.BlockSpec((1, 1, 1, BN_ROWS), lambda b, j: (b, j, 0, 0)),
        ],
        out_specs=[
            pl.BlockSpec((1, BN_ROWS, 64), lambda b, j: (b, j, 0)),
            pl.BlockSpec((2, 64), lambda b, j: (0, 0)),
        ],
        out_shape=[
            jax.ShapeDtypeStruct((B, NP_PAD, 64), jnp.float32),
            jax.ShapeDtypeStruct((2, 64), jnp.float32),
        ],
        scratch_shapes=[pltpu.VMEM((2, 64), jnp.float32)],
    )(pool, wt, e4, ks4)
    gb = jnp.stack([bn_g, bn_b], axis=0)
    y, yemp = pl.pallas_call(
        functools.partial(_conv_norm_body, nv=nv),
        grid=(B, N_BLOCKS),
        in_specs=[
            pl.BlockSpec((1, BN_ROWS, 64), lambda b, j: (b, j, 0)),
            pl.BlockSpec((2, 64), lambda b, j: (0, 0)),
            pl.BlockSpec((2, 64), lambda b, j: (0, 0)),
        ],
        out_specs=[
            pl.BlockSpec((1, BN_ROWS, 64), lambda b, j: (b, j, 0)),
            pl.BlockSpec((1, 64), lambda b, j: (0, 0)),
        ],
        out_shape=[
            jax.ShapeDtypeStruct((B, NP_PAD, 64), jnp.float32),
            jax.ShapeDtypeStruct((1, 64), jnp.float32),
        ],
    )(vc, stats, gb)
    return y, yemp


def _bilerp_body(i_ref, w_ref, out_ref):
    ii = i_ref[0]
    w = w_ref[0, 0]
    out_ref[0] = (ii[:, 0, :] * w[:, 0:1] + ii[:, 1, :] * w[:, 1:2]
                  + ii[:, 2, :] * w[:, 2:3] + ii[:, 3, :] * w[:, 3:4])


def _bilerp(ivals, w4):
    return pl.pallas_call(
        _bilerp_body,
        grid=(B, N_BLOCKS),
        in_specs=[
            pl.BlockSpec((1, BN_ROWS, 4, 64), lambda b, j: (b, j, 0, 0)),
            pl.BlockSpec((1, 1, BN_ROWS, 4), lambda b, j: (b, j, 0, 0)),
        ],
        out_specs=pl.BlockSpec((1, BN_ROWS, 64), lambda b, j: (b, j, 0)),
        out_shape=jax.ShapeDtypeStruct((B, NP_PAD, 64), jnp.float32),
    )(ivals, w4)


def _round_up(x, m):
    return (x + m - 1) // m * m


def _view_precompute(p_planes, mask_pl, grid, ranges):
    gx, gy = grid
    nv = gx * gy
    s_stride = nv + 2
    s_len = _round_up(B * s_stride, NS * 16 * CH)
    key_pl, valid_pl, cen_pl, pay_pl, idxg_pl, ids4_pl, w4_pl = _prep(
        p_planes, mask_pl, grid, ranges)
    key = key_pl.reshape(B, NP_PAD)
    iota = jnp.broadcast_to(jnp.arange(NP_PAD, dtype=jnp.int32), (B, NP_PAD))
    ks, perm = jax.vmap(lambda k, i: lax.sort((k, i), num_keys=1))(key, iota)
    kp = jnp.concatenate([jnp.full((B, 1), -1, jnp.int32), ks[:, :-1]], 1)
    kn = jnp.concatenate([ks[:, 1:], jnp.full((B, 1), -1, jnp.int32)], 1)
    ks4 = ks.reshape(B, N_BLOCKS, 1, BN_ROWS)
    kp4 = kp.reshape(B, N_BLOCKS, 1, BN_ROWS)
    kn4 = kn.reshape(B, N_BLOCKS, 1, BN_ROWS)
    permg = (perm + (jnp.arange(B, dtype=jnp.int32) * NP_PAD)[:, None]
             ).reshape(M_PTS)
    pay = pay_pl.transpose(0, 1, 3, 4, 2).reshape(M_PTS, 16)
    pay_sorted = _sc_gather_rows(pay, permg)
    tot, e4, sval4, scell4 = _seg_scan(ks4, kp4, kn4,
                                       pay_sorted.reshape(B, NP_PAD, 16),
                                       is_max=False, with_slots=True,
                                       s_stride=s_stride, nv=nv)
    s_map = _sc_scatter_add(scell4.reshape(M_PTS), sval4.reshape(M_PTS),
                            s_len)
    stats_table = jnp.concatenate(
        [jnp.zeros((N_EMPTY, 16), jnp.float32), tot.reshape(M_PTS, 16)],
        axis=0)
    gstats = _sc_gather2(s_map, stats_table, idxg_pl.reshape(M_PTS))
    g_planes = gstats.reshape(B, N_BLOCKS, R24, 128, 16).transpose(
        0, 1, 4, 2, 3)
    pview_pl = _stats_finalize(g_planes, p_planes, valid_pl, mask_pl, cen_pl)
    ids4 = ids4_pl.transpose(0, 1, 3, 4, 2).reshape(M_PTS * 4)
    w4 = w4_pl.transpose(0, 1, 3, 4, 2).reshape(B, N_BLOCKS, BN_ROWS, 4)
    return dict(nv=nv, s_stride=s_stride, s_map=s_map, ks4=ks4, kp4=kp4,
                kn4=kn4, e4=e4, permg=permg, ids4=ids4, w4=w4,
                pview_pl=pview_pl)


def _single_view(x_pad, mask4, vc, grid, params):
    nv = vc['nv']
    h = _pointnet(x_pad, mask4, params['pn'])
    h_sorted = _sc_gather_rows(h.reshape(M_PTS, 64), vc['permg'])
    pool = _seg_scan(vc['ks4'], vc['kp4'], vc['kn4'],
                     h_sorted.reshape(B, NP_PAD, 64), is_max=True,
                     with_slots=False)
    y, yemp = _conv_bn(pool, params['conv_w'], params['bn_g'],
                       params['bn_b'], vc['e4'], vc['ks4'], nv)
    img_table = jnp.concatenate(
        [jnp.broadcast_to(yemp, (N_EMPTY, 64)), y.reshape(M_PTS, 64)], axis=0)
    ivals = _sc_gather2(vc['s_map'], img_table, vc['ids4'])
    return _bilerp(ivals.reshape(B, NP_PAD, 4, 64), vc['w4'])


def _to_planes(x_pad):
    return x_pad.reshape(B, N_BLOCKS, R24, 128, x_pad.shape[-1]).transpose(
        0, 1, 4, 2, 3)


def kernel(points_xyz, points_feature, points_mask, params):
    p_pad = _pad_pts(points_xyz)
    mask_pad = _pad_pts(points_mask)
    mask4 = mask_pad.reshape(B, N_BLOCKS, 1, BN_ROWS)
    maskp = _pad_pts(points_mask, value=-1.0)
    mask_pl = maskp.reshape(B, N_BLOCKS, R24, 128)
    x = points_xyz
    rho = jnp.sqrt(x[..., 0] ** 2 + x[..., 1] ** 2)
    theta = jnp.arctan2(x[..., 1], x[..., 0])
    pc = jnp.stack([theta, x[..., 2], rho], axis=-1)
    pc_pad = _pad_pts(pc)
    p_planes = _to_planes(p_pad)
    pc_planes = _to_planes(pc_pad)

    xyc = _view_precompute(p_planes, mask_pl, XY_GRID, XY_R)
    cyc = _view_precompute(pc_planes, mask_pl, CYL_GRID, CYL_R)

    feat_pl = jnp.concatenate(
        [p_planes, xyc['pview_pl'], pc_planes, cyc['pview_pl'],
         _pad_pts(points_feature).reshape(B, N_BLOCKS, 1, R24, 128)], axis=2)
    feat = feat_pl.transpose(0, 1, 3, 4, 2).reshape(B, NP_PAD, 45)
    feat = _pad_to(feat, 128, 2)
    x1 = _pointnet(feat, mask4, params['pn1'])
    x1p = _pad_to(x1, 128, 2)
    xxy = _single_view(x1p, mask4, xyc, XY_GRID, params['xy'])
    xcy = _single_view(x1p, mask4, cyc, CYL_GRID, params['cyl'])
    xpw = _pointnet(x1p, mask4, params['pn2'])
    x2 = jnp.concatenate([xxy, xcy, xpw], axis=-1)
    x2 = _pad_to(x2, 256, 2)
    x3 = _pointnet(x2, mask4, params['pn3'])

    x3_sorted = _sc_gather_rows(x3.reshape(M_PTS, 64), xyc['permg'])
    pool = _seg_scan(xyc['ks4'], xyc['kp4'], xyc['kn4'],
                     x3_sorted.reshape(B, NP_PAD, 64), is_max=True,
                     with_slots=False)
    nv = xyc['nv']
    ks = xyc['ks4'].reshape(B, NP_PAD)
    e = xyc['e4'].reshape(B, NP_PAD) > 0.5
    real = e & (ks < nv)
    boff = (jnp.arange(B, dtype=jnp.int32) * nv)[:, None]
    cell = jnp.where(real, ks + boff, B * nv).reshape(M_PTS)
    grid_flat = jnp.zeros((B * nv, 64), jnp.float32)
    grid_flat = grid_flat.at[cell].set(pool.reshape(M_PTS, 64),
                                       mode='drop',
                                       unique_indices=True)
    pil = grid_flat.reshape(B, XY_GRID[0], XY_GRID[1], 64)
    return pil.transpose(0, 3, 2, 1)
